# Initial kernel scaffold; baseline (speedup 1.0000x reference)
import jax
import jax.numpy as jnp
from jax.experimental import pallas as pl
from jax.experimental.pallas import tpu as pltpu
from jax.experimental.pallas import tpu_sc as plsc

_GWS = 128


def _embedding_gather(idx_flat, table):
    n_idx = idx_flat.shape[1]
    dim = table.shape[1]
    mesh = plsc.VectorSubcoreMesh(core_axis_name="c", subcore_axis_name="s")

    @pl.kernel(
        out_type=jax.ShapeDtypeStruct((n_idx, dim), table.dtype),
        mesh=mesh,
    )
    def kern(table_hbm, idx_hbm, out_hbm):
        def body(i_vmem, o_vmem):
            pltpu.sync_copy(table_hbm.at[i_vmem.at[0]], o_vmem)

        pltpu.emit_pipeline(
            body,
            grid=(n_idx // _GWS,),
            in_specs=[pl.BlockSpec((1, _GWS), lambda i: (0, i))],
            out_specs=[pl.BlockSpec((_GWS, dim), lambda i: (i, 0))],
            core_axis_name=("c", "s"),
            dimension_semantics=(pltpu.PARALLEL,),
        )(idx_hbm, out_hbm)

    return kern(table, idx_flat)


def kernel(inputs, table):
    batch, hist = inputs.shape
    idx_flat = inputs.astype(jnp.int32).reshape(1, batch * hist)
    out = _embedding_gather(idx_flat, table)
    return out.reshape(batch, hist, table.shape[1])


# speedup vs baseline: 1.4189x; 1.4189x over previous
import jax
import jax.numpy as jnp
from jax.experimental import pallas as pl
from jax.experimental.pallas import tpu as pltpu
from jax.experimental.pallas import tpu_sc as plsc

_GWS = 128


def _embedding_gather(idx_flat, table):
    n_idx = idx_flat.shape[1]
    dim = table.shape[1]
    mesh = plsc.VectorSubcoreMesh(core_axis_name="c", subcore_axis_name="s")

    @pl.kernel(
        out_type=jax.ShapeDtypeStruct((n_idx, dim), table.dtype),
        mesh=mesh,
        compiler_params=pltpu.CompilerParams(use_tc_tiling_on_sc=False),
    )
    def kern(table_hbm, idx_hbm, out_hbm):
        def body(i_vmem, o_vmem):
            pltpu.sync_copy(table_hbm.at[i_vmem.at[0]], o_vmem)

        pltpu.emit_pipeline(
            body,
            grid=(n_idx // _GWS,),
            in_specs=[pl.BlockSpec((1, _GWS), lambda i: (0, i))],
            out_specs=[pl.BlockSpec((_GWS, dim), lambda i: (i, 0))],
            core_axis_name=("c", "s"),
            dimension_semantics=(pltpu.PARALLEL,),
        )(idx_hbm, out_hbm)

    return kern(table, idx_flat)


def kernel(inputs, table):
    batch, hist = inputs.shape
    idx_flat = inputs.astype(jnp.int32).reshape(1, batch * hist)
    out = _embedding_gather(idx_flat, table)
    return out.reshape(batch, hist, table.shape[1])


# speedup vs baseline: 1.5740x; 1.1093x over previous
import jax
import jax.numpy as jnp
from jax.experimental import pallas as pl
from jax.experimental.pallas import tpu as pltpu
from jax.experimental.pallas import tpu_sc as plsc

_GWS = 1024


def _embedding_gather(idx_flat, table):
    n_idx = idx_flat.shape[1]
    dim = table.shape[1]
    mesh = plsc.VectorSubcoreMesh(core_axis_name="c", subcore_axis_name="s")

    @pl.kernel(
        out_type=jax.ShapeDtypeStruct((n_idx, dim), table.dtype),
        mesh=mesh,
        compiler_params=pltpu.CompilerParams(use_tc_tiling_on_sc=False),
    )
    def kern(table_hbm, idx_hbm, out_hbm):
        def body(i_vmem, o_vmem):
            pltpu.sync_copy(table_hbm.at[i_vmem.at[0]], o_vmem)

        pltpu.emit_pipeline(
            body,
            grid=(n_idx // _GWS,),
            in_specs=[pl.BlockSpec((1, _GWS), lambda i: (0, i))],
            out_specs=[pl.BlockSpec((_GWS, dim), lambda i: (i, 0))],
            core_axis_name=("c", "s"),
            dimension_semantics=(pltpu.PARALLEL,),
        )(idx_hbm, out_hbm)

    return kern(table, idx_flat)


def kernel(inputs, table):
    batch, hist = inputs.shape
    idx_flat = inputs.astype(jnp.int32).reshape(1, batch * hist)
    out = _embedding_gather(idx_flat, table)
    return out.reshape(batch, hist, table.shape[1])


# speedup vs baseline: 1.6502x; 1.0484x over previous
import jax
import jax.numpy as jnp
from jax.experimental import pallas as pl
from jax.experimental.pallas import tpu as pltpu
from jax.experimental.pallas import tpu_sc as plsc

_GWS = 1024


def _embedding_gather(idx_flat, table):
    n_idx = idx_flat.shape[1]
    dim = table.shape[1]
    mesh = plsc.VectorSubcoreMesh(core_axis_name="c", subcore_axis_name="s")

    @pl.kernel(
        out_type=jax.ShapeDtypeStruct((n_idx, dim), table.dtype),
        mesh=mesh,
        compiler_params=pltpu.CompilerParams(use_tc_tiling_on_sc=False),
    )
    def kern(table_hbm, idx_hbm, out_hbm):
        def body(i_vmem, o_vmem):
            pltpu.sync_copy(table_hbm.at[i_vmem.at[0]], o_vmem)

        pltpu.emit_pipeline(
            body,
            grid=(n_idx // _GWS,),
            in_specs=[pl.BlockSpec((1, _GWS), lambda i: (0, i))],
            out_specs=[pl.BlockSpec((_GWS, dim), lambda i: (i, 0))],
            core_axis_name=("c", "s"),
            dimension_semantics=(pltpu.PARALLEL,),
        )(idx_hbm, out_hbm)

    return kern(table, idx_flat)


def kernel(inputs, table):
    batch, hist = inputs.shape
    idx_flat = jnp.transpose(inputs).astype(jnp.int32).reshape(1, batch * hist)
    out = _embedding_gather(idx_flat, table)
    return jnp.transpose(out.reshape(hist, batch, table.shape[1]), (1, 0, 2))
